# baseline (speedup 1.0000x reference)
import functools

import jax
import jax.numpy as jnp
from jax import lax
from jax.experimental import pallas as pl
from jax.experimental.pallas import tpu as pltpu
from jax.experimental.pallas import tpu_sc as plsc

_N = 10000
_D = 128
_E = 320000
_NSC = 2
_NSUB = 16
_CS = 128
_CH = 160
_NPH = 10
_HCH = _CH // _NPH
_PW = _CH * _CS
_EPAD = _NSUB * _PW
_AGG_ROWS = 10240
_ZROWS = _AGG_ROWS // _NSUB


def _sc_agg_body(h_hbm, src_hbm, dst_hbm, zeros_hbm, out_hbm,
                 src_v0, dst_v0, src_v1, dst_v1, gb0, gb1, agg_sh,
                 gsem0, gsem1, isem):
    c = lax.axis_index("c")
    s = lax.axis_index("s")
    gbufs = (gb0, gb1)
    gsems = (gsem0, gsem1)
    ibufs = ((src_v0, dst_v0), (src_v1, dst_v1))

    def idx_copies(phase, pb):
        sv, dv = ibufs[pb]
        off = pl.ds(phase * _HCH, _HCH)
        return ((src_hbm.at[c, s, off], sv), (dst_hbm.at[c, s, off], dv))

    def issue_gather(j, b, sv):
        pltpu.async_copy(h_hbm.at[c].at[sv.at[j]], gbufs[b], gsems[b])

    def wait_gather(j, b, sv):
        pltpu.make_async_copy(h_hbm.at[c].at[sv.at[j]], gbufs[b],
                              gsems[b]).wait()

    for src, dst in idx_copies(0, 0):
        pltpu.sync_copy(src, dst)
    issue_gather(0, 0, src_v0)
    issue_gather(1, 1, src_v0)
    pltpu.sync_copy(zeros_hbm.at[pl.ds(s * _ZROWS, _ZROWS)],
                    agg_sh.at[pl.ds(s * _ZROWS, _ZROWS)])
    plsc.subcore_barrier()

    for phase in range(_NPH):
        pb = phase % 2
        sv, dv = ibufs[pb]
        if phase + 1 < _NPH:
            for src, dst in idx_copies(phase + 1, 1 - pb):
                pltpu.async_copy(src, dst, isem)

        def body(jj, carry):
            for b in range(2):
                j = 2 * jj + b
                wait_gather(j, b, sv)
                pltpu.sync_copy(gbufs[b], agg_sh.at[dv.at[j]], add=True)
                nxt = j + 2

                @pl.when(nxt < _HCH)
                def _():
                    issue_gather(nxt, b, sv)
            return carry

        lax.fori_loop(0, _HCH // 2, body, 0)

        if phase + 1 < _NPH:
            for src, dst in idx_copies(phase + 1, 1 - pb):
                pltpu.make_async_copy(src, dst, isem).wait()
            nsv = ibufs[1 - pb][0]
            issue_gather(0, 0, nsv)
            issue_gather(1, 1, nsv)

    plsc.subcore_barrier()
    pltpu.sync_copy(agg_sh.at[pl.ds(s * _ZROWS, _ZROWS)],
                    out_hbm.at[c, pl.ds(s * _ZROWS, _ZROWS)])


@jax.jit
def _sc_agg(h_stack, src_p, dst_p, zeros):
    k = pl.kernel(
        _sc_agg_body,
        out_type=jax.ShapeDtypeStruct((_NSC, _AGG_ROWS, _D), jnp.float32),
        mesh=plsc.VectorSubcoreMesh(core_axis_name="c", subcore_axis_name="s"),
        scratch_types=(
            [pltpu.VMEM((_HCH, _CS), jnp.int32)] * 4
            + [pltpu.VMEM((_CS, _D), jnp.float32)] * 2
            + [pltpu.VMEM_SHARED((_AGG_ROWS, _D), jnp.float32)]
            + [pltpu.SemaphoreType.DMA] * 3
        ),
    )
    return k(h_stack, src_p, dst_p, zeros)


def _bn_body(xs_ref, is_ref, xt_ref, it_ref, g_ref, b_ref, o_ref):
    for gi, (xr, ir) in enumerate(((xs_ref, is_ref), (xt_ref, it_ref))):
        y = xr[...] * ir[...]
        m = jnp.mean(y, axis=0, keepdims=True)
        d = y - m
        v = jnp.mean(d * d, axis=0, keepdims=True)
        o_ref[gi] = d * lax.rsqrt(v + 1e-5) * g_ref[...] + b_ref[...]


def _bn(xs, imps, xt, impt, g, b):
    return pl.pallas_call(
        _bn_body,
        out_shape=jax.ShapeDtypeStruct((_NSC, _N, _D), jnp.float32),
    )(xs, imps, xt, impt, g.reshape(1, _D), b.reshape(1, _D))


def _mlp_body(h_ref, a_ref, w_ref, b_ref, o_ref):
    x = (h_ref[...] + a_ref[:, :_N]).reshape(_NSC * _N, _D)
    o_ref[...] = jnp.tanh(
        jnp.dot(x, w_ref[...], preferred_element_type=jnp.float32)
        + b_ref[...]).reshape(_NSC, _N, _D)


def _mlp(h, agg, w, b):
    return pl.pallas_call(
        _mlp_body,
        out_shape=jax.ShapeDtypeStruct((_NSC, _N, _D), jnp.float32),
    )(h, agg, w, b.reshape(1, _D))


def _mlp_fc_body(h_ref, a_ref, w_ref, b_ref, wfc_ref, o5_ref, o6_ref):
    x = (h_ref[...] + a_ref[:, :_N]).reshape(_NSC * _N, _D)
    h5 = jnp.tanh(
        jnp.dot(x, w_ref[...], preferred_element_type=jnp.float32)
        + b_ref[...])
    o5_ref[...] = h5.reshape(_NSC, _N, _D)
    o6_ref[...] = jnp.tanh(
        jnp.dot(h5, wfc_ref[...],
                preferred_element_type=jnp.float32)).reshape(_NSC, _N, _D)


def _mlp_fc(h, agg, w, b, wfc):
    return pl.pallas_call(
        _mlp_fc_body,
        out_shape=(jax.ShapeDtypeStruct((_NSC, _N, _D), jnp.float32),
                   jax.ShapeDtypeStruct((_NSC, _N, _D), jnp.float32)),
    )(h, agg, w, b.reshape(1, _D), wfc)


def _pad_edges(ei):
    pad = _EPAD - _E
    ar = jnp.arange(pad, dtype=jnp.int32)
    src = jnp.concatenate([ei[0], ar % _N]).reshape(_NSUB, _CH, _CS)
    dst = jnp.concatenate(
        [ei[1], _N + (ar % (_AGG_ROWS - _N))]).reshape(_NSUB, _CH, _CS)
    return src, dst


def kernel(source_x, source_x_importance, source_edge_index, target_x,
           target_x_importance, target_edge_index, bn_gamma, bn_beta,
           W1, b1, W2, b2, W3, b3, W4, b4, W5, b5, Wfc):
    Ws = [W1, W2, W3, W4, W5]
    bs = [b1, b2, b3, b4, b5]
    zeros = jnp.zeros((_AGG_ROWS, _D), jnp.float32)
    ss, sd = _pad_edges(source_edge_index)
    ts, td = _pad_edges(target_edge_index)
    src_p = jnp.stack([ss, ts])
    dst_p = jnp.stack([sd, td])
    h = _bn(source_x, source_x_importance, target_x, target_x_importance,
            bn_gamma, bn_beta)
    hs = []
    for i, (W, bb) in enumerate(zip(Ws, bs)):
        agg = _sc_agg(h, src_p, dst_p, zeros)
        if i < 4:
            h = _mlp(h, agg, W, bb)
            hs.append(h)
        else:
            h5, h6 = _mlp_fc(h, agg, W, bb, Wfc)
            hs.append(h5)
            hs.append(h6)
    out_s = jnp.concatenate([hh[0] for hh in hs], axis=-1)
    out_t = jnp.concatenate([hh[1] for hh in hs], axis=-1)
    return (out_s, out_t)


# speedup vs baseline: 1.0327x; 1.0327x over previous
import functools

import jax
import jax.numpy as jnp
from jax import lax
from jax.experimental import pallas as pl
from jax.experimental.pallas import tpu as pltpu
from jax.experimental.pallas import tpu_sc as plsc

_N = 10000
_D = 128
_E = 320000
_NSC = 2
_NSUB = 16
_CS = 128
_CH = 160
_NPH = 5
_HCH = _CH // _NPH
_PW = _CH * _CS
_EPAD = _NSUB * _PW
_AGG_ROWS = 10112
_ZROWS = _AGG_ROWS // _NSUB


def _sc_agg_body(h_hbm, src_hbm, dst_hbm, zeros_hbm, out_hbm,
                 src_v0, dst_v0, src_v1, dst_v1, gb0, gb1, agg_sh,
                 gsem0, gsem1, isem):
    c = lax.axis_index("c")
    s = lax.axis_index("s")
    gbufs = (gb0, gb1)
    gsems = (gsem0, gsem1)
    ibufs = ((src_v0, dst_v0), (src_v1, dst_v1))

    def idx_copies(phase, pb):
        sv, dv = ibufs[pb]
        off = pl.ds(phase * _HCH, _HCH)
        return ((src_hbm.at[c, s, off], sv), (dst_hbm.at[c, s, off], dv))

    def issue_gather(j, b, sv):
        pltpu.async_copy(h_hbm.at[c].at[sv.at[j]], gbufs[b], gsems[b])

    def wait_gather(j, b, sv):
        pltpu.make_async_copy(h_hbm.at[c].at[sv.at[j]], gbufs[b],
                              gsems[b]).wait()

    for src, dst in idx_copies(0, 0):
        pltpu.sync_copy(src, dst)
    issue_gather(0, 0, src_v0)
    issue_gather(1, 1, src_v0)
    pltpu.sync_copy(zeros_hbm.at[pl.ds(s * _ZROWS, _ZROWS)],
                    agg_sh.at[pl.ds(s * _ZROWS, _ZROWS)])
    plsc.subcore_barrier()

    for phase in range(_NPH):
        pb = phase % 2
        sv, dv = ibufs[pb]
        if phase + 1 < _NPH:
            for src, dst in idx_copies(phase + 1, 1 - pb):
                pltpu.async_copy(src, dst, isem)

        def body(jj, carry):
            for b in range(2):
                j = 2 * jj + b
                wait_gather(j, b, sv)
                pltpu.sync_copy(gbufs[b], agg_sh.at[dv.at[j]], add=True)
                nxt = j + 2

                @pl.when(nxt < _HCH)
                def _():
                    issue_gather(nxt, b, sv)
            return carry

        lax.fori_loop(0, _HCH // 2, body, 0)

        if phase + 1 < _NPH:
            for src, dst in idx_copies(phase + 1, 1 - pb):
                pltpu.make_async_copy(src, dst, isem).wait()
            nsv = ibufs[1 - pb][0]
            issue_gather(0, 0, nsv)
            issue_gather(1, 1, nsv)

    plsc.subcore_barrier()
    pltpu.sync_copy(agg_sh.at[pl.ds(s * _ZROWS, _ZROWS)],
                    out_hbm.at[c, pl.ds(s * _ZROWS, _ZROWS)])


@jax.jit
def _sc_agg(h_stack, src_p, dst_p, zeros):
    k = pl.kernel(
        _sc_agg_body,
        out_type=jax.ShapeDtypeStruct((_NSC, _AGG_ROWS, _D), jnp.float32),
        mesh=plsc.VectorSubcoreMesh(core_axis_name="c", subcore_axis_name="s"),
        scratch_types=(
            [pltpu.VMEM((_HCH, _CS), jnp.int32)] * 4
            + [pltpu.VMEM((_CS, _D), jnp.float32)] * 2
            + [pltpu.VMEM_SHARED((_AGG_ROWS, _D), jnp.float32)]
            + [pltpu.SemaphoreType.DMA] * 3
        ),
    )
    return k(h_stack, src_p, dst_p, zeros)


def _bn_body(xs_ref, is_ref, xt_ref, it_ref, g_ref, b_ref, o_ref):
    for gi, (xr, ir) in enumerate(((xs_ref, is_ref), (xt_ref, it_ref))):
        y = xr[...] * ir[...]
        m = jnp.mean(y, axis=0, keepdims=True)
        d = y - m
        v = jnp.mean(d * d, axis=0, keepdims=True)
        o_ref[gi] = d * lax.rsqrt(v + 1e-5) * g_ref[...] + b_ref[...]


def _bn(xs, imps, xt, impt, g, b):
    return pl.pallas_call(
        _bn_body,
        out_shape=jax.ShapeDtypeStruct((_NSC, _N, _D), jnp.float32),
    )(xs, imps, xt, impt, g.reshape(1, _D), b.reshape(1, _D))


def _mlp_body(h_ref, a_ref, w_ref, b_ref, o_ref):
    x = (h_ref[...] + a_ref[:, :_N]).reshape(_NSC * _N, _D)
    o_ref[...] = jnp.tanh(
        jnp.dot(x, w_ref[...], preferred_element_type=jnp.float32)
        + b_ref[...]).reshape(_NSC, _N, _D)


def _mlp(h, agg, w, b):
    return pl.pallas_call(
        _mlp_body,
        out_shape=jax.ShapeDtypeStruct((_NSC, _N, _D), jnp.float32),
    )(h, agg, w, b.reshape(1, _D))


def _mlp_fc_body(h_ref, a_ref, w_ref, b_ref, wfc_ref, o5_ref, o6_ref):
    x = (h_ref[...] + a_ref[:, :_N]).reshape(_NSC * _N, _D)
    h5 = jnp.tanh(
        jnp.dot(x, w_ref[...], preferred_element_type=jnp.float32)
        + b_ref[...])
    o5_ref[...] = h5.reshape(_NSC, _N, _D)
    o6_ref[...] = jnp.tanh(
        jnp.dot(h5, wfc_ref[...],
                preferred_element_type=jnp.float32)).reshape(_NSC, _N, _D)


def _mlp_fc(h, agg, w, b, wfc):
    return pl.pallas_call(
        _mlp_fc_body,
        out_shape=(jax.ShapeDtypeStruct((_NSC, _N, _D), jnp.float32),
                   jax.ShapeDtypeStruct((_NSC, _N, _D), jnp.float32)),
    )(h, agg, w, b.reshape(1, _D), wfc)


def _pad_edges(ei):
    pad = _EPAD - _E
    ar = jnp.arange(pad, dtype=jnp.int32)
    src = jnp.concatenate([ei[0], ar % _N]).reshape(_NSUB, _CH, _CS)
    dst = jnp.concatenate(
        [ei[1], _N + (ar % (_AGG_ROWS - _N))]).reshape(_NSUB, _CH, _CS)
    return src, dst


def kernel(source_x, source_x_importance, source_edge_index, target_x,
           target_x_importance, target_edge_index, bn_gamma, bn_beta,
           W1, b1, W2, b2, W3, b3, W4, b4, W5, b5, Wfc):
    Ws = [W1, W2, W3, W4, W5]
    bs = [b1, b2, b3, b4, b5]
    zeros = jnp.zeros((_AGG_ROWS, _D), jnp.float32)
    ss, sd = _pad_edges(source_edge_index)
    ts, td = _pad_edges(target_edge_index)
    src_p = jnp.stack([ss, ts])
    dst_p = jnp.stack([sd, td])
    h = _bn(source_x, source_x_importance, target_x, target_x_importance,
            bn_gamma, bn_beta)
    hs = []
    for i, (W, bb) in enumerate(zip(Ws, bs)):
        agg = _sc_agg(h, src_p, dst_p, zeros)
        if i < 4:
            h = _mlp(h, agg, W, bb)
            hs.append(h)
        else:
            h5, h6 = _mlp_fc(h, agg, W, bb, Wfc)
            hs.append(h5)
            hs.append(h6)
    out_s = jnp.concatenate([hh[0] for hh in hs], axis=-1)
    out_t = jnp.concatenate([hh[1] for hh in hs], axis=-1)
    return (out_s, out_t)
